# baseline (speedup 1.0000x reference)
import jax
import jax.numpy as jnp
from jax import lax
from jax.experimental import pallas as pl
from jax.experimental.pallas import tpu as pltpu
from jax.experimental.pallas import tpu_sc as plsc

N = 10000
NP = 10240
HALF = 64
TILES = 16
BLK = 2048
IDXW = 128
RPB = BLK // IDXW
ZROWS = 128
K = 1
NPAR = 4
NG = RPB // K

E1, E2 = 320000, 640000
PT1 = -(-E1 // (TILES * BLK)) * BLK
PT2 = -(-E2 // (TILES * BLK)) * BLK
NB1 = PT1 // BLK
NB2 = PT2 // BLK
R1TOT = TILES * NB1 * RPB
DUMMY = N


def _sc_body(xstack, sidx, didx, zeros_hbm, out,
             rows, sbuf, dbuf, acc, xcache,
             gs0, gs1, gs2, gs3, ss0, ss1, ss2, ss3, isem):
    c = lax.axis_index("c")
    s = lax.axis_index("s")
    gsem = (gs0, gs1, gs2, gs3)
    ssem = (ss0, ss1, ss2, ss3)
    hbm_tab = xstack.at[c]
    def tbl(g):
        return xcache if (g % 8) < 5 else hbm_tab

    def tile_prog(base, nblocks):
        def groups(b, par, nxt, first_block):
            for gi in range(NG):
                p = gi % NPAR
                np_ = (gi + 2) % NPAR
                for u in range(K):
                    pltpu.make_async_copy(
                        tbl(gi).at[sbuf.at[0, 0]], rows.at[p * K + u],
                        gsem[p]).wait()
                for u in range(K):
                    pltpu.async_copy(rows.at[p * K + u],
                                     acc.at[dbuf.at[par, gi * K + u]],
                                     ssem[p], add=True)
                if not (first_block and gi < 2):
                    for u in range(K):
                        pltpu.make_async_copy(
                            rows.at[np_ * K + u], acc.at[dbuf.at[0, 0]],
                            ssem[np_]).wait()
                if gi == 2 and not first_block:
                    @pl.when(b + 1 < nblocks)
                    def _():
                        r0 = base + (b + 1) * RPB
                        pltpu.async_copy(sidx.at[pl.ds(r0, RPB)],
                                         sbuf.at[nxt], isem)
                        pltpu.async_copy(didx.at[pl.ds(r0, RPB)],
                                         dbuf.at[nxt], isem)
                if gi < NG - 2:
                    for u in range(K):
                        pltpu.async_copy(
                            tbl(gi + 2).at[sbuf.at[par, (gi + 2) * K + u]],
                            rows.at[np_ * K + u], gsem[np_])
                else:
                    def fire_next(gi=gi, np_=np_):
                        if gi == NG - 2:
                            pltpu.make_async_copy(
                                sidx.at[pl.ds(base, RPB)],
                                sbuf.at[nxt], isem).wait()
                            pltpu.make_async_copy(
                                didx.at[pl.ds(base, RPB)],
                                dbuf.at[nxt], isem).wait()
                        g2 = gi - (NG - 2)
                        for u in range(K):
                            pltpu.async_copy(
                                tbl(g2).at[sbuf.at[nxt, g2 * K + u]],
                                rows.at[np_ * K + u], gsem[np_])
                    if first_block:
                        fire_next()
                    else:
                        pl.when(b + 1 < nblocks)(fire_next)

        pltpu.sync_copy(sidx.at[pl.ds(base, RPB)], sbuf.at[0])
        pltpu.sync_copy(didx.at[pl.ds(base, RPB)], dbuf.at[0])
        pltpu.async_copy(sidx.at[pl.ds(base + RPB, RPB)], sbuf.at[1], isem)
        pltpu.async_copy(didx.at[pl.ds(base + RPB, RPB)], dbuf.at[1], isem)
        for g in range(2):
            for u in range(K):
                pltpu.async_copy(tbl(g).at[sbuf.at[0, g * K + u]],
                                 rows.at[g * K + u], gsem[g])

        groups(0, 0, 1, True)

        def block(b, carry):
            par = lax.rem(b, 2)
            groups(b, par, 1 - par, False)
            return carry

        lax.fori_loop(1, nblocks, block, 0)

        for p in (2, 3):
            for u in range(K):
                pltpu.make_async_copy(rows.at[p * K + u],
                                      acc.at[dbuf.at[0, 0]], ssem[p]).wait()

    span = NP // TILES
    zb = pl.multiple_of(s * span, ZROWS)
    pltpu.sync_copy(xstack.at[c, pl.ds(zb, span)], xcache.at[pl.ds(zb, span)])
    ob = pl.multiple_of(s * span, 8)
    for l, (rbase, nblocks) in enumerate(((0, NB1), (R1TOT, NB2))):
        pltpu.sync_copy(zeros_hbm, rows.at[0])
        for k in range(span // ZROWS):
            pltpu.sync_copy(rows.at[0], acc.at[pl.ds(zb + k * ZROWS, ZROWS)])
        plsc.subcore_barrier()

        tile_prog(pl.multiple_of(rbase + s * nblocks * RPB, 8), nblocks)

        plsc.subcore_barrier()
        pltpu.sync_copy(acc.at[pl.ds(ob, span)], out.at[l, c, pl.ds(ob, span)])


@jax.jit
def kernel(x, adj_t, adj_t2):
    xstack = jnp.pad(x.reshape(N, 2, HALF).transpose(1, 0, 2),
                     ((0, 0), (0, NP - N), (0, 0)))

    def pad_idx(src, dst, per_tile, e):
        p = TILES * per_tile - e
        src = jnp.concatenate([src, jnp.zeros((p,), jnp.int32)])
        dst = jnp.concatenate([dst, jnp.full((p,), DUMMY, jnp.int32)])
        return src.reshape(-1, IDXW), dst.reshape(-1, IDXW)

    s1, d1 = pad_idx(adj_t[0], adj_t[1], PT1, E1)
    s2, d2 = pad_idx(adj_t2[0], adj_t2[1], PT2, E2)
    sidx = jnp.concatenate([s1, s2], axis=0)
    didx = jnp.concatenate([d1, d2], axis=0)
    zeros = jnp.zeros((ZROWS, HALF), jnp.float32)

    mesh = plsc.VectorSubcoreMesh(core_axis_name="c", subcore_axis_name="s",
                                  num_cores=2, num_subcores=TILES)
    run = pl.kernel(
        _sc_body,
        out_type=jax.ShapeDtypeStruct((2, 2, NP, HALF), jnp.float32),
        mesh=mesh,
        scratch_types=[
            pltpu.VMEM((NPAR * K, IDXW, HALF), jnp.float32),
            pltpu.VMEM((2, RPB, IDXW), jnp.int32),
            pltpu.VMEM((2, RPB, IDXW), jnp.int32),
            pltpu.VMEM_SHARED((NP, HALF), jnp.float32),
            pltpu.VMEM_SHARED((NP, HALF), jnp.float32),
        ] + [pltpu.SemaphoreType.DMA] * 9,
        compiler_params=pltpu.CompilerParams(use_tc_tiling_on_sc=False),
    )
    out = run(xstack, sidx, didx, zeros)
    return out[:, :, :N].transpose(2, 0, 1, 3).reshape(N, 4 * HALF)


# speedup vs baseline: 1.4118x; 1.4118x over previous
import jax
import jax.numpy as jnp
from jax import lax
from jax.experimental import pallas as pl
from jax.experimental.pallas import tpu as pltpu
from jax.experimental.pallas import tpu_sc as plsc

N = 10000
NP = 10240
HALF = 64
TILES = 16
BLK = 2048
IDXW = 64
RPB = BLK // IDXW
ZROWS = IDXW
K = 1
NPAR = 8
AHEAD = NPAR // 2
NG = RPB // K

E1, E2 = 320000, 640000
PT1 = -(-E1 // (TILES * BLK)) * BLK
PT2 = -(-E2 // (TILES * BLK)) * BLK
NB1 = PT1 // BLK
NB2 = PT2 // BLK
R1TOT = TILES * NB1 * RPB
DUMMY = N


def _sc_body(xstack, sidx, didx, zeros_hbm, out,
             rows, sbuf, dbuf, acc, xcache, gsem, ssem, isem):
    c = lax.axis_index("c")
    s = lax.axis_index("s")
    table = xcache

    def tile_prog(base, nblocks):
        def groups(b, par, nxt, first_block):
            for gi in range(NG):
                p = gi % NPAR
                np_ = (gi + AHEAD) % NPAR
                for u in range(K):
                    pltpu.make_async_copy(
                        table.at[sbuf.at[0, 0]], rows.at[p * K + u],
                        gsem[p]).wait()
                for u in range(K):
                    pltpu.async_copy(rows.at[p * K + u],
                                     acc.at[dbuf.at[par, gi * K + u]],
                                     ssem[p], add=True)
                if not (first_block and gi < AHEAD):
                    for u in range(K):
                        pltpu.make_async_copy(
                            rows.at[np_ * K + u], acc.at[dbuf.at[0, 0]],
                            ssem[np_]).wait()
                if gi == AHEAD and not first_block:
                    @pl.when(b + 1 < nblocks)
                    def _():
                        r0 = base + (b + 1) * RPB
                        pltpu.async_copy(sidx.at[pl.ds(r0, RPB)],
                                         sbuf.at[nxt], isem)
                        pltpu.async_copy(didx.at[pl.ds(r0, RPB)],
                                         dbuf.at[nxt], isem)
                if gi < NG - AHEAD:
                    for u in range(K):
                        pltpu.async_copy(
                            table.at[sbuf.at[par, (gi + AHEAD) * K + u]],
                            rows.at[np_ * K + u], gsem[np_])
                else:
                    def fire_next(gi=gi, np_=np_):
                        if gi == NG - AHEAD:
                            pltpu.make_async_copy(
                                sidx.at[pl.ds(base, RPB)],
                                sbuf.at[nxt], isem).wait()
                            pltpu.make_async_copy(
                                didx.at[pl.ds(base, RPB)],
                                dbuf.at[nxt], isem).wait()
                        g2 = gi - (NG - AHEAD)
                        for u in range(K):
                            pltpu.async_copy(
                                table.at[sbuf.at[nxt, g2 * K + u]],
                                rows.at[np_ * K + u], gsem[np_])
                    if first_block:
                        fire_next()
                    else:
                        pl.when(b + 1 < nblocks)(fire_next)

        pltpu.sync_copy(sidx.at[pl.ds(base, RPB)], sbuf.at[0])
        pltpu.sync_copy(didx.at[pl.ds(base, RPB)], dbuf.at[0])
        pltpu.async_copy(sidx.at[pl.ds(base + RPB, RPB)], sbuf.at[1], isem)
        pltpu.async_copy(didx.at[pl.ds(base + RPB, RPB)], dbuf.at[1], isem)
        for g in range(AHEAD):
            for u in range(K):
                pltpu.async_copy(table.at[sbuf.at[0, g * K + u]],
                                 rows.at[g * K + u], gsem[g])

        groups(0, 0, 1, True)

        def block(b, carry):
            par = lax.rem(b, 2)
            groups(b, par, 1 - par, False)
            return carry

        lax.fori_loop(1, nblocks, block, 0)

        for k in range(AHEAD):
            p = (NG - AHEAD + k) % NPAR
            for u in range(K):
                pltpu.make_async_copy(rows.at[p * K + u],
                                      acc.at[dbuf.at[0, 0]], ssem[p]).wait()

    span = NP // TILES
    zb = pl.multiple_of(s * span, ZROWS)
    pltpu.sync_copy(xstack.at[c, pl.ds(zb, span)], xcache.at[pl.ds(zb, span)])
    ob = pl.multiple_of(s * span, 8)
    for l, (rbase, nblocks) in enumerate(((0, NB1), (R1TOT, NB2))):
        pltpu.sync_copy(zeros_hbm, rows.at[0])
        for k in range(span // ZROWS):
            pltpu.sync_copy(rows.at[0], acc.at[pl.ds(zb + k * ZROWS, ZROWS)])
        plsc.subcore_barrier()

        tile_prog(pl.multiple_of(rbase + s * nblocks * RPB, 8), nblocks)

        plsc.subcore_barrier()
        pltpu.sync_copy(acc.at[pl.ds(ob, span)], out.at[l, c, pl.ds(ob, span)])


@jax.jit
def kernel(x, adj_t, adj_t2):
    xstack = jnp.pad(x.reshape(N, 2, HALF).transpose(1, 0, 2),
                     ((0, 0), (0, NP - N), (0, 0)))

    def pad_idx(src, dst, per_tile, e):
        p = TILES * per_tile - e
        src = jnp.concatenate([src, jnp.zeros((p,), jnp.int32)])
        dst = jnp.concatenate([dst, jnp.full((p,), DUMMY, jnp.int32)])
        return src.reshape(-1, IDXW), dst.reshape(-1, IDXW)

    s1, d1 = pad_idx(adj_t[0], adj_t[1], PT1, E1)
    s2, d2 = pad_idx(adj_t2[0], adj_t2[1], PT2, E2)
    sidx = jnp.concatenate([s1, s2], axis=0)
    didx = jnp.concatenate([d1, d2], axis=0)
    zeros = jnp.zeros((ZROWS, HALF), jnp.float32)

    mesh = plsc.VectorSubcoreMesh(core_axis_name="c", subcore_axis_name="s",
                                  num_cores=2, num_subcores=TILES)
    run = pl.kernel(
        _sc_body,
        out_type=jax.ShapeDtypeStruct((2, 2, NP, HALF), jnp.float32),
        mesh=mesh,
        scratch_types=[
            pltpu.VMEM((NPAR * K, IDXW, HALF), jnp.float32),
            pltpu.VMEM((2, RPB, IDXW), jnp.int32),
            pltpu.VMEM((2, RPB, IDXW), jnp.int32),
            pltpu.VMEM_SHARED((NP, HALF), jnp.float32),
            pltpu.VMEM_SHARED((NP, HALF), jnp.float32),
            [pltpu.SemaphoreType.DMA] * NPAR,
            [pltpu.SemaphoreType.DMA] * NPAR,
            pltpu.SemaphoreType.DMA,
        ],
        compiler_params=pltpu.CompilerParams(use_tc_tiling_on_sc=False),
    )
    out = run(xstack, sidx, didx, zeros)
    return out[:, :, :N].transpose(2, 0, 1, 3).reshape(N, 4 * HALF)


# speedup vs baseline: 1.5164x; 1.0741x over previous
import jax
import jax.numpy as jnp
from jax import lax
from jax.experimental import pallas as pl
from jax.experimental.pallas import tpu as pltpu
from jax.experimental.pallas import tpu_sc as plsc

N = 10000
NP = 10240
HALF = 64
TILES = 16
BLK = 2048
IDXW = 128
RPB = BLK // IDXW
ZROWS = 128
K = 1
NPAR = 4
NG = RPB // K

E1, E2 = 320000, 640000
PT1 = -(-E1 // (TILES * BLK)) * BLK
PT2 = -(-E2 // (TILES * BLK)) * BLK
NB1 = PT1 // BLK
NB2 = PT2 // BLK
R1TOT = TILES * NB1 * RPB
DUMMY = N


def _sc_body(xstack, sidx, didx, zeros_hbm, out,
             rows, sbuf, dbuf, acc, xcache,
             gs0, gs1, gs2, gs3, ss0, ss1, ss2, ss3, isem):
    c = lax.axis_index("c")
    s = lax.axis_index("s")
    gsem = (gs0, gs1, gs2, gs3)
    ssem = (ss0, ss1, ss2, ss3)
    table = xcache

    def tile_prog(base, nblocks):
        def groups(b, par, nxt, first_block):
            for gi in range(NG):
                p = gi % NPAR
                np_ = (gi + 3) % NPAR
                for u in range(K):
                    pltpu.make_async_copy(
                        table.at[sbuf.at[0, 0]], rows.at[p * K + u],
                        gsem[p]).wait()
                for u in range(K):
                    pltpu.async_copy(rows.at[p * K + u],
                                     acc.at[dbuf.at[par, gi * K + u]],
                                     ssem[p], add=True)
                if not (first_block and gi < 1):
                    for u in range(K):
                        pltpu.make_async_copy(
                            rows.at[np_ * K + u], acc.at[dbuf.at[0, 0]],
                            ssem[np_]).wait()
                if gi == 2 and not first_block:
                    @pl.when(b + 1 < nblocks)
                    def _():
                        r0 = base + (b + 1) * RPB
                        pltpu.async_copy(sidx.at[pl.ds(r0, RPB)],
                                         sbuf.at[nxt], isem)
                        pltpu.async_copy(didx.at[pl.ds(r0, RPB)],
                                         dbuf.at[nxt], isem)
                if gi < NG - 3:
                    for u in range(K):
                        pltpu.async_copy(
                            table.at[sbuf.at[par, (gi + 3) * K + u]],
                            rows.at[np_ * K + u], gsem[np_])
                else:
                    def fire_next(gi=gi, np_=np_):
                        if gi == NG - 3:
                            pltpu.make_async_copy(
                                sidx.at[pl.ds(base, RPB)],
                                sbuf.at[nxt], isem).wait()
                            pltpu.make_async_copy(
                                didx.at[pl.ds(base, RPB)],
                                dbuf.at[nxt], isem).wait()
                        for u in range(K):
                            pltpu.async_copy(
                                table.at[sbuf.at[nxt, (gi - (NG - 3)) * K + u]],
                                rows.at[np_ * K + u], gsem[np_])
                    if first_block:
                        fire_next()
                    else:
                        pl.when(b + 1 < nblocks)(fire_next)

        pltpu.sync_copy(sidx.at[pl.ds(base, RPB)], sbuf.at[0])
        pltpu.sync_copy(didx.at[pl.ds(base, RPB)], dbuf.at[0])
        pltpu.async_copy(sidx.at[pl.ds(base + RPB, RPB)], sbuf.at[1], isem)
        pltpu.async_copy(didx.at[pl.ds(base + RPB, RPB)], dbuf.at[1], isem)
        for g in range(3):
            for u in range(K):
                pltpu.async_copy(table.at[sbuf.at[0, g * K + u]],
                                 rows.at[g * K + u], gsem[g])

        groups(0, 0, 1, True)

        def block(b, carry):
            par = lax.rem(b, 2)
            groups(b, par, 1 - par, False)
            return carry

        lax.fori_loop(1, nblocks, block, 0)

        for p in (3,):
            for u in range(K):
                pltpu.make_async_copy(rows.at[p * K + u],
                                      acc.at[dbuf.at[0, 0]], ssem[p]).wait()

    span = NP // TILES
    zb = pl.multiple_of(s * span, ZROWS)
    pltpu.sync_copy(xstack.at[c, pl.ds(zb, span)], xcache.at[pl.ds(zb, span)])
    ob = pl.multiple_of(s * span, 8)
    for l, (rbase, nblocks) in enumerate(((0, NB1), (R1TOT, NB2))):
        pltpu.sync_copy(zeros_hbm, rows.at[0])
        for k in range(span // ZROWS):
            pltpu.sync_copy(rows.at[0], acc.at[pl.ds(zb + k * ZROWS, ZROWS)])
        plsc.subcore_barrier()

        tile_prog(pl.multiple_of(rbase + s * nblocks * RPB, 8), nblocks)

        plsc.subcore_barrier()
        pltpu.sync_copy(acc.at[pl.ds(ob, span)], out.at[l, c, pl.ds(ob, span)])


@jax.jit
def kernel(x, adj_t, adj_t2):
    xstack = jnp.pad(x.reshape(N, 2, HALF).transpose(1, 0, 2),
                     ((0, 0), (0, NP - N), (0, 0)))

    def pad_idx(src, dst, per_tile, e):
        p = TILES * per_tile - e
        src = jnp.concatenate([src, jnp.zeros((p,), jnp.int32)])
        dst = jnp.concatenate([dst, jnp.full((p,), DUMMY, jnp.int32)])
        return src.reshape(-1, IDXW), dst.reshape(-1, IDXW)

    s1, d1 = pad_idx(adj_t[0], adj_t[1], PT1, E1)
    s2, d2 = pad_idx(adj_t2[0], adj_t2[1], PT2, E2)
    sidx = jnp.concatenate([s1, s2], axis=0)
    didx = jnp.concatenate([d1, d2], axis=0)
    zeros = jnp.zeros((ZROWS, HALF), jnp.float32)

    mesh = plsc.VectorSubcoreMesh(core_axis_name="c", subcore_axis_name="s",
                                  num_cores=2, num_subcores=TILES)
    run = pl.kernel(
        _sc_body,
        out_type=jax.ShapeDtypeStruct((2, 2, NP, HALF), jnp.float32),
        mesh=mesh,
        scratch_types=[
            pltpu.VMEM((NPAR * K, IDXW, HALF), jnp.float32),
            pltpu.VMEM((2, RPB, IDXW), jnp.int32),
            pltpu.VMEM((2, RPB, IDXW), jnp.int32),
            pltpu.VMEM_SHARED((NP, HALF), jnp.float32),
            pltpu.VMEM_SHARED((NP, HALF), jnp.float32),
        ] + [pltpu.SemaphoreType.DMA] * 9,
        compiler_params=pltpu.CompilerParams(use_tc_tiling_on_sc=False),
    )
    out = run(xstack, sidx, didx, zeros)
    return out[:, :, :N].transpose(2, 0, 1, 3).reshape(N, 4 * HALF)
